# baseline (speedup 1.0000x reference)
import functools
import math
import jax
import jax.numpy as jnp
from jax import lax
from jax.experimental import pallas as pl
from jax.experimental.pallas import tpu as pltpu
from jax.experimental.pallas import tpu_sc as plsc

_UB = 8


def _prologue(enc_ref, wqa_ref, bqa_ref, embs_ref, wqs_ref, bqs_ref,
              wk_ref, bk_ref, dec_ref, qac_ref, qse_ref, kv_ref):
    V1, Dh = embs_ref.shape
    V = V1 - 1
    A = wk_ref.shape[1]
    BU = dec_ref.shape[0]
    embs = embs_ref[...]
    qac_ref[...] = (jnp.dot(enc_ref[...], wqa_ref[...],
                            preferred_element_type=jnp.float32)
                    + bqa_ref[...])
    kv_ref[:, :A] = (jnp.dot(embs, wk_ref[...],
                             preferred_element_type=jnp.float32)
                     + bk_ref[...])
    kv_ref[:, A:] = embs
    onehot = (dec_ref[...] ==
              jax.lax.broadcasted_iota(jnp.int32, (BU, V), 1)
              ).astype(jnp.float32)
    semantic = jnp.dot(onehot, embs[:V, :], preferred_element_type=jnp.float32)
    qse_ref[...] = (jnp.dot(semantic, wqs_ref[...],
                            preferred_element_type=jnp.float32)
                    + bqs_ref[...])


def _sc_gather_body(kv_hbm, idx_hbm, out_hbm, idx_v, rows_v, sem,
                    *, pairs_per_worker):
    nc = 2
    wid = lax.axis_index("s") * nc + lax.axis_index("c")
    for i in range(pairs_per_worker):
        p = wid * pairs_per_worker + i
        pltpu.sync_copy(idx_hbm.at[p], idx_v)
        pltpu.async_copy(kv_hbm.at[idx_v], rows_v, sem).wait()
        pltpu.sync_copy(rows_v, out_hbm.at[p])


def _main(qac_ref, qse_ref, idxr_ref, idxc_ref, kvg_ref, wd_ref, bd_ref,
          ptr_ref, h_ref, db_ref):
    C = idxr_ref.shape[2]
    V1 = ptr_ref.shape[3]
    A = qac_ref.shape[2]
    inv_sqrt_a = 1.0 / math.sqrt(A)
    qac = qac_ref[0]
    wd = wd_ref[...]
    bd = bd_ref[...]
    iota_v = jax.lax.broadcasted_iota(jnp.int32, (C, V1), 1)
    iota_r = jax.lax.broadcasted_iota(jnp.int32, (C, C), 0)
    iota_c = jax.lax.broadcasted_iota(jnp.int32, (C, C), 1)
    for j in range(_UB):
        idx_col = idxc_ref[0, j]
        idx_row = idxr_ref[0, j:j + 1, :]
        G = (idx_col == iota_v).astype(jnp.float32)
        eq = idx_col == idx_row
        has_later = jnp.any(eq & (iota_c > iota_r), axis=1,
                            keepdims=True)
        S = jnp.where(has_later, 0.0, G)
        kvg = kvg_ref[j][:C, :]
        keys = kvg[:, :A]
        values = kvg[:, A:]
        q = qac + qse_ref[0, j:j + 1, :]
        logits = jax.lax.dot_general(
            q, keys, (((1,), (1,)), ((), ())),
            preferred_element_type=jnp.float32) * inv_sqrt_a
        logits = jnp.where(idx_row < 0, -1.0e9, logits)
        m = jnp.max(logits, axis=1, keepdims=True)
        e = jnp.exp(logits - m)
        atten = e / jnp.sum(e, axis=1, keepdims=True)
        x = jnp.dot(atten, values, preferred_element_type=jnp.float32)
        h_ref[0, :, j, :] = x
        db_ref[0, :, j, :] = jnp.dot(x, wd,
                                     preferred_element_type=jnp.float32) + bd
        ptr_ref[0, :, j, :] = jnp.dot(atten, S,
                                      preferred_element_type=jnp.float32)


def kernel(encoder_out, decoder_in, masks_mat, dec_embed_weight, ooKB_weight,
           Wqa, bqa, Wqs, bqs, Wk, bk, Wd, bd):
    B, T, Eh = encoder_out.shape
    U = decoder_in.shape[1]
    C = masks_mat.shape[2]
    V, Dh = dec_embed_weight.shape
    A = Wk.shape[1]
    J = Wd.shape[1]
    V1 = V + 1
    KVW = A + Dh
    f32 = jnp.float32

    embs = jnp.concatenate([dec_embed_weight, ooKB_weight], axis=0)
    enc2d = encoder_out.reshape(B * T, Eh)
    dec2d = decoder_in.reshape(B * U, 1).astype(jnp.int32)

    qac2d, qse2d, kv = pl.pallas_call(
        _prologue,
        out_shape=(
            jax.ShapeDtypeStruct((B * T, A), f32),
            jax.ShapeDtypeStruct((B * U, A), f32),
            jax.ShapeDtypeStruct((V1, KVW), f32),
        ),
    )(enc2d, Wqa, bqa.reshape(1, A), embs, Wqs, bqs.reshape(1, A),
      Wk, bk.reshape(1, A), dec2d)

    qac3 = qac2d.reshape(B, T, A)
    qse3 = qse2d.reshape(B, U, A)
    masks_row = masks_mat.astype(jnp.int32)
    masks_col = masks_row[..., None]

    nu = pl.cdiv(U, _UB)
    u_pad = nu * _UB
    n_pairs = B * u_pad
    n_workers = 32
    assert n_pairs % n_workers == 0
    ppw = n_pairs // n_workers
    cp = pl.cdiv(C, 8) * 8
    idx_pairs = jnp.pad(masks_row, ((0, 0), (0, u_pad - U), (0, cp - C))
                        ).reshape(n_pairs, cp)

    mesh = plsc.VectorSubcoreMesh(core_axis_name="c", subcore_axis_name="s")
    sc_gather = functools.partial(
        pl.kernel,
        mesh=mesh,
        out_type=jax.ShapeDtypeStruct((n_pairs, cp, KVW), f32),
        scratch_types=[
            pltpu.VMEM((cp,), jnp.int32),
            pltpu.VMEM((cp, KVW), f32),
            pltpu.SemaphoreType.DMA,
        ],
    )(functools.partial(_sc_gather_body, pairs_per_worker=ppw))
    kvg = sc_gather(kv, idx_pairs)

    grid = (B, nu)
    ptr, h_ptr, dbias = pl.pallas_call(
        _main,
        grid=grid,
        in_specs=[
            pl.BlockSpec((1, T, A), lambda b, u: (b, 0, 0)),
            pl.BlockSpec((1, _UB, A), lambda b, u: (b, u, 0)),
            pl.BlockSpec((1, _UB, C), lambda b, u: (b, u, 0)),
            pl.BlockSpec((1, _UB, C, 1), lambda b, u: (b, u, 0, 0)),
            pl.BlockSpec((_UB, cp, KVW), lambda b, u, nu=nu: (b * nu + u, 0, 0)),
            pl.BlockSpec((Dh, J), lambda b, u: (0, 0)),
            pl.BlockSpec((1, J), lambda b, u: (0, 0)),
        ],
        out_specs=[
            pl.BlockSpec((1, T, _UB, V1), lambda b, u: (b, 0, u, 0)),
            pl.BlockSpec((1, T, _UB, Dh), lambda b, u: (b, 0, u, 0)),
            pl.BlockSpec((1, T, _UB, J), lambda b, u: (b, 0, u, 0)),
        ],
        out_shape=(
            jax.ShapeDtypeStruct((B, T, U, V1), f32),
            jax.ShapeDtypeStruct((B, T, U, Dh), f32),
            jax.ShapeDtypeStruct((B, T, U, J), f32),
        ),
    )(qac3, qse3, masks_row, masks_col, kvg, Wd, bd.reshape(1, J))

    return (ptr, h_ptr, dbias)


# speedup vs baseline: 1.0112x; 1.0112x over previous
import functools
import math
import jax
import jax.numpy as jnp
from jax import lax
from jax.experimental import pallas as pl
from jax.experimental.pallas import tpu as pltpu
from jax.experimental.pallas import tpu_sc as plsc

_UB = 8


def _prologue(enc_ref, wqa_ref, bqa_ref, embs_ref, wqs_ref, bqs_ref,
              wk_ref, bk_ref, dec_ref, qac_ref, qse_ref, kv_ref):
    V1, Dh = embs_ref.shape
    V = V1 - 1
    A = wk_ref.shape[1]
    BU = dec_ref.shape[0]
    embs = embs_ref[...]
    qac_ref[...] = (jnp.dot(enc_ref[...], wqa_ref[...],
                            preferred_element_type=jnp.float32)
                    + bqa_ref[...])
    kv_ref[:, :A] = (jnp.dot(embs, wk_ref[...],
                             preferred_element_type=jnp.float32)
                     + bk_ref[...])
    kv_ref[:, A:] = embs
    onehot = (dec_ref[...] ==
              jax.lax.broadcasted_iota(jnp.int32, (BU, V), 1)
              ).astype(jnp.float32)
    semantic = jnp.dot(onehot, embs[:V, :], preferred_element_type=jnp.float32)
    qse_ref[...] = (jnp.dot(semantic, wqs_ref[...],
                            preferred_element_type=jnp.float32)
                    + bqs_ref[...])


def _sc_gather_body(kv_hbm, idx_hbm, out_hbm, idx_all, rows0, rows1,
                    gs0, gs1, ws0, ws1, *, pairs_per_worker):
    nc = 2
    ppw = pairs_per_worker
    wid = lax.axis_index("s") * nc + lax.axis_index("c")
    base = wid * ppw
    pltpu.sync_copy(idx_hbm.at[wid], idx_all)
    bufs = (rows0, rows1)
    gsems = (gs0, gs1)
    wsems = (ws0, ws1)
    gh = [None] * ppw
    wh = [None] * ppw
    for i in range(ppw):
        if i >= 2:
            wh[i - 2].wait()
        gh[i] = pltpu.async_copy(kv_hbm.at[idx_all.at[i]], bufs[i % 2],
                                 gsems[i % 2])
        if i >= 1:
            gh[i - 1].wait()
            wh[i - 1] = pltpu.async_copy(bufs[(i - 1) % 2],
                                         out_hbm.at[base + i - 1],
                                         wsems[(i - 1) % 2])
    gh[ppw - 1].wait()
    wh[ppw - 1] = pltpu.async_copy(bufs[(ppw - 1) % 2],
                                   out_hbm.at[base + ppw - 1],
                                   wsems[(ppw - 1) % 2])
    if ppw >= 2:
        wh[ppw - 2].wait()
    wh[ppw - 1].wait()


def _main(qac_ref, qse_ref, idxr_ref, idxc_ref, kvg_ref, wd_ref, bd_ref,
          ptr_ref, h_ref, db_ref):
    C = idxr_ref.shape[2]
    V1 = ptr_ref.shape[3]
    A = qac_ref.shape[2]
    inv_sqrt_a = 1.0 / math.sqrt(A)
    qac = qac_ref[0]
    wd = wd_ref[...]
    bd = bd_ref[...]
    iota_v = jax.lax.broadcasted_iota(jnp.int32, (C, V1), 1)
    iota_r = jax.lax.broadcasted_iota(jnp.int32, (C, C), 0)
    iota_c = jax.lax.broadcasted_iota(jnp.int32, (C, C), 1)
    for j in range(_UB):
        idx_col = idxc_ref[0, j]
        idx_row = idxr_ref[0, j:j + 1, :]
        G = (idx_col == iota_v).astype(jnp.float32)
        eq = idx_col == idx_row
        has_later = jnp.any(eq & (iota_c > iota_r), axis=1,
                            keepdims=True)
        S = jnp.where(has_later, 0.0, G)
        kvg = kvg_ref[j][:C, :]
        keys = kvg[:, :A]
        values = kvg[:, A:]
        q = qac + qse_ref[0, j:j + 1, :]
        logits = jax.lax.dot_general(
            q, keys, (((1,), (1,)), ((), ())),
            preferred_element_type=jnp.float32) * inv_sqrt_a
        logits = jnp.where(idx_row < 0, -1.0e9, logits)
        m = jnp.max(logits, axis=1, keepdims=True)
        e = jnp.exp(logits - m)
        atten = e / jnp.sum(e, axis=1, keepdims=True)
        x = jnp.dot(atten, values, preferred_element_type=jnp.float32)
        h_ref[0, :, j, :] = x
        db_ref[0, :, j, :] = jnp.dot(x, wd,
                                     preferred_element_type=jnp.float32) + bd
        ptr_ref[0, :, j, :] = jnp.dot(atten, S,
                                      preferred_element_type=jnp.float32)


def kernel(encoder_out, decoder_in, masks_mat, dec_embed_weight, ooKB_weight,
           Wqa, bqa, Wqs, bqs, Wk, bk, Wd, bd):
    B, T, Eh = encoder_out.shape
    U = decoder_in.shape[1]
    C = masks_mat.shape[2]
    V, Dh = dec_embed_weight.shape
    A = Wk.shape[1]
    J = Wd.shape[1]
    V1 = V + 1
    KVW = A + Dh
    f32 = jnp.float32

    embs = jnp.concatenate([dec_embed_weight, ooKB_weight], axis=0)
    enc2d = encoder_out.reshape(B * T, Eh)
    dec2d = decoder_in.reshape(B * U, 1).astype(jnp.int32)

    qac2d, qse2d, kv = pl.pallas_call(
        _prologue,
        out_shape=(
            jax.ShapeDtypeStruct((B * T, A), f32),
            jax.ShapeDtypeStruct((B * U, A), f32),
            jax.ShapeDtypeStruct((V1, KVW), f32),
        ),
    )(enc2d, Wqa, bqa.reshape(1, A), embs, Wqs, bqs.reshape(1, A),
      Wk, bk.reshape(1, A), dec2d)

    qac3 = qac2d.reshape(B, T, A)
    qse3 = qse2d.reshape(B, U, A)
    masks_row = masks_mat.astype(jnp.int32)
    masks_col = masks_row[..., None]

    nu = pl.cdiv(U, _UB)
    u_pad = nu * _UB
    n_pairs = B * u_pad
    n_workers = 32
    assert n_pairs % n_workers == 0
    ppw = n_pairs // n_workers
    cp = pl.cdiv(C, 8) * 8
    idx_pairs = jnp.pad(masks_row, ((0, 0), (0, u_pad - U), (0, cp - C))
                        ).reshape(n_workers, ppw, cp)

    mesh = plsc.VectorSubcoreMesh(core_axis_name="c", subcore_axis_name="s")
    sc_gather = functools.partial(
        pl.kernel,
        mesh=mesh,
        out_type=jax.ShapeDtypeStruct((n_pairs, cp, KVW), f32),
        scratch_types=[
            pltpu.VMEM((ppw, cp), jnp.int32),
            pltpu.VMEM((cp, KVW), f32),
            pltpu.VMEM((cp, KVW), f32),
            pltpu.SemaphoreType.DMA,
            pltpu.SemaphoreType.DMA,
            pltpu.SemaphoreType.DMA,
            pltpu.SemaphoreType.DMA,
        ],
    )(functools.partial(_sc_gather_body, pairs_per_worker=ppw))
    kvg = sc_gather(kv, idx_pairs)

    grid = (B, nu)
    ptr, h_ptr, dbias = pl.pallas_call(
        _main,
        grid=grid,
        in_specs=[
            pl.BlockSpec((1, T, A), lambda b, u: (b, 0, 0)),
            pl.BlockSpec((1, _UB, A), lambda b, u: (b, u, 0)),
            pl.BlockSpec((1, _UB, C), lambda b, u: (b, u, 0)),
            pl.BlockSpec((1, _UB, C, 1), lambda b, u: (b, u, 0, 0)),
            pl.BlockSpec((_UB, cp, KVW), lambda b, u, nu=nu: (b * nu + u, 0, 0)),
            pl.BlockSpec((Dh, J), lambda b, u: (0, 0)),
            pl.BlockSpec((1, J), lambda b, u: (0, 0)),
        ],
        out_specs=[
            pl.BlockSpec((1, T, _UB, V1), lambda b, u: (b, 0, u, 0)),
            pl.BlockSpec((1, T, _UB, Dh), lambda b, u: (b, 0, u, 0)),
            pl.BlockSpec((1, T, _UB, J), lambda b, u: (b, 0, u, 0)),
        ],
        out_shape=(
            jax.ShapeDtypeStruct((B, T, U, V1), f32),
            jax.ShapeDtypeStruct((B, T, U, Dh), f32),
            jax.ShapeDtypeStruct((B, T, U, J), f32),
        ),
    )(qac3, qse3, masks_row, masks_col, kvg, Wd, bd.reshape(1, J))

    return (ptr, h_ptr, dbias)


# speedup vs baseline: 1.0257x; 1.0144x over previous
import functools
import math
import jax
import jax.numpy as jnp
from jax import lax
from jax.experimental import pallas as pl
from jax.experimental.pallas import tpu as pltpu
from jax.experimental.pallas import tpu_sc as plsc

_UB = 8


def _prologue(enc_ref, wqa_ref, bqa_ref, embs_ref, wqs_ref, bqs_ref,
              dec_ref, qac_ref, qse_ref):
    V1, Dh = embs_ref.shape
    V = V1 - 1
    BU = dec_ref.shape[0]
    embs = embs_ref[...]
    qac_ref[...] = (jnp.dot(enc_ref[...], wqa_ref[...],
                            preferred_element_type=jnp.float32)
                    + bqa_ref[...])
    onehot = (dec_ref[...] ==
              jax.lax.broadcasted_iota(jnp.int32, (BU, V), 1)
              ).astype(jnp.float32)
    semantic = jnp.dot(onehot, embs[:V, :], preferred_element_type=jnp.float32)
    qse_ref[...] = (jnp.dot(semantic, wqs_ref[...],
                            preferred_element_type=jnp.float32)
                    + bqs_ref[...])


def _sc_gather_body(tab_hbm, idx_hbm, out_hbm, idx_all, rows0, rows1,
                    gs0, gs1, ws0, ws1, *, pairs_per_worker):
    nc = 2
    ppw = pairs_per_worker
    wid = lax.axis_index("s") * nc + lax.axis_index("c")
    base = wid * ppw
    pltpu.sync_copy(idx_hbm.at[wid], idx_all)
    bufs = (rows0, rows1)
    gsems = (gs0, gs1)
    wsems = (ws0, ws1)
    gh = [None] * ppw
    wh = [None] * ppw
    for i in range(ppw):
        if i >= 2:
            wh[i - 2].wait()
        gh[i] = pltpu.async_copy(tab_hbm.at[idx_all.at[i]], bufs[i % 2],
                                 gsems[i % 2])
        if i >= 1:
            gh[i - 1].wait()
            wh[i - 1] = pltpu.async_copy(bufs[(i - 1) % 2],
                                         out_hbm.at[base + i - 1],
                                         wsems[(i - 1) % 2])
    gh[ppw - 1].wait()
    wh[ppw - 1] = pltpu.async_copy(bufs[(ppw - 1) % 2],
                                   out_hbm.at[base + ppw - 1],
                                   wsems[(ppw - 1) % 2])
    if ppw >= 2:
        wh[ppw - 2].wait()
    wh[ppw - 1].wait()


def _main(qac_ref, qse_ref, idxr_ref, idxc_ref, vg_ref, wk_ref, bk_ref,
          wd_ref, bd_ref, ptr_ref, h_ref, db_ref):
    C = idxr_ref.shape[2]
    V1 = ptr_ref.shape[3]
    A = qac_ref.shape[2]
    bf16 = jnp.bfloat16
    inv_sqrt_a = 1.0 / math.sqrt(A)
    qac = qac_ref[0]
    wk = wk_ref[...].astype(bf16)
    bk = bk_ref[...]
    wd = wd_ref[...].astype(bf16)
    bd = bd_ref[...]
    iota_v = jax.lax.broadcasted_iota(jnp.int32, (C, V1), 1)
    iota_r = jax.lax.broadcasted_iota(jnp.int32, (C, C), 0)
    iota_c = jax.lax.broadcasted_iota(jnp.int32, (C, C), 1)
    for j in range(_UB):
        idx_col = idxc_ref[0, j]
        idx_row = idxr_ref[0, j:j + 1, :]
        G = (idx_col == iota_v).astype(bf16)
        eq = idx_col == idx_row
        has_later = jnp.any(eq & (iota_c > iota_r), axis=1,
                            keepdims=True)
        S = jnp.where(has_later, bf16(0.0), G)
        values = vg_ref[j][:C, :]
        values_b = values.astype(bf16)
        keys = jnp.dot(values_b, wk,
                       preferred_element_type=jnp.float32) + bk
        q = (qac + qse_ref[0, j:j + 1, :]).astype(bf16)
        logits = jax.lax.dot_general(
            q, keys.astype(bf16), (((1,), (1,)), ((), ())),
            preferred_element_type=jnp.float32) * inv_sqrt_a
        logits = jnp.where(idx_row < 0, -1.0e9, logits)
        m = jnp.max(logits, axis=1, keepdims=True)
        e = jnp.exp(logits - m)
        atten = e / jnp.sum(e, axis=1, keepdims=True)
        atten_b = atten.astype(bf16)
        x = jnp.dot(atten_b, values_b, preferred_element_type=jnp.float32)
        h_ref[0, :, j, :] = x
        db_ref[0, :, j, :] = jnp.dot(x.astype(bf16), wd,
                                     preferred_element_type=jnp.float32) + bd
        ptr_ref[0, :, j, :] = jnp.dot(atten_b, S,
                                      preferred_element_type=jnp.float32)


def kernel(encoder_out, decoder_in, masks_mat, dec_embed_weight, ooKB_weight,
           Wqa, bqa, Wqs, bqs, Wk, bk, Wd, bd):
    B, T, Eh = encoder_out.shape
    U = decoder_in.shape[1]
    C = masks_mat.shape[2]
    V, Dh = dec_embed_weight.shape
    A = Wk.shape[1]
    J = Wd.shape[1]
    V1 = V + 1
    f32 = jnp.float32

    embs = jnp.concatenate([dec_embed_weight, ooKB_weight], axis=0)
    enc2d = encoder_out.reshape(B * T, Eh)
    dec2d = decoder_in.reshape(B * U, 1).astype(jnp.int32)

    qac2d, qse2d = pl.pallas_call(
        _prologue,
        out_shape=(
            jax.ShapeDtypeStruct((B * T, A), f32),
            jax.ShapeDtypeStruct((B * U, A), f32),
        ),
    )(enc2d, Wqa, bqa.reshape(1, A), embs, Wqs, bqs.reshape(1, A), dec2d)

    qac3 = qac2d.reshape(B, T, A)
    qse3 = qse2d.reshape(B, U, A)
    masks_row = masks_mat.astype(jnp.int32)
    masks_col = masks_row[..., None]

    nu = pl.cdiv(U, _UB)
    u_pad = nu * _UB
    n_pairs = B * u_pad
    n_workers = 32
    assert n_pairs % n_workers == 0
    ppw = n_pairs // n_workers
    cp = pl.cdiv(C, 8) * 8
    idx_pairs = jnp.pad(masks_row, ((0, 0), (0, u_pad - U), (0, cp - C))
                        ).reshape(n_workers, ppw, cp)

    mesh = plsc.VectorSubcoreMesh(core_axis_name="c", subcore_axis_name="s")
    sc_gather = functools.partial(
        pl.kernel,
        mesh=mesh,
        out_type=jax.ShapeDtypeStruct((n_pairs, cp, Dh), f32),
        scratch_types=[
            pltpu.VMEM((ppw, cp), jnp.int32),
            pltpu.VMEM((cp, Dh), f32),
            pltpu.VMEM((cp, Dh), f32),
            pltpu.SemaphoreType.DMA,
            pltpu.SemaphoreType.DMA,
            pltpu.SemaphoreType.DMA,
            pltpu.SemaphoreType.DMA,
        ],
    )(functools.partial(_sc_gather_body, pairs_per_worker=ppw))
    vg = sc_gather(embs, idx_pairs)

    grid = (B, nu)
    ptr, h_ptr, dbias = pl.pallas_call(
        _main,
        grid=grid,
        in_specs=[
            pl.BlockSpec((1, T, A), lambda b, u: (b, 0, 0)),
            pl.BlockSpec((1, _UB, A), lambda b, u: (b, u, 0)),
            pl.BlockSpec((1, _UB, C), lambda b, u: (b, u, 0)),
            pl.BlockSpec((1, _UB, C, 1), lambda b, u: (b, u, 0, 0)),
            pl.BlockSpec((_UB, cp, Dh), lambda b, u, nu=nu: (b * nu + u, 0, 0)),
            pl.BlockSpec((Dh, A), lambda b, u: (0, 0)),
            pl.BlockSpec((1, A), lambda b, u: (0, 0)),
            pl.BlockSpec((Dh, J), lambda b, u: (0, 0)),
            pl.BlockSpec((1, J), lambda b, u: (0, 0)),
        ],
        out_specs=[
            pl.BlockSpec((1, T, _UB, V1), lambda b, u: (b, 0, u, 0)),
            pl.BlockSpec((1, T, _UB, Dh), lambda b, u: (b, 0, u, 0)),
            pl.BlockSpec((1, T, _UB, J), lambda b, u: (b, 0, u, 0)),
        ],
        out_shape=(
            jax.ShapeDtypeStruct((B, T, U, V1), f32),
            jax.ShapeDtypeStruct((B, T, U, Dh), f32),
            jax.ShapeDtypeStruct((B, T, U, J), f32),
        ),
    )(qac3, qse3, masks_row, masks_col, vg, Wk, bk.reshape(1, A),
      Wd, bd.reshape(1, J))

    return (ptr, h_ptr, dbias)


# speedup vs baseline: 1.7020x; 1.6593x over previous
import functools
import math
import jax
import jax.numpy as jnp
from jax.experimental import pallas as pl

_UB = 8


def _main(enc_ref, dec_ref, idxp_ref, idxc_ref, embs_ref,
          wqa_ref, bqa_ref, wqs_ref, bqs_ref, wk_ref, bk_ref,
          wd_ref, bd_ref, ptr_ref, h_ref, db_ref, *, C):
    T = enc_ref.shape[1]
    cp = idxp_ref.shape[2]
    V1, Dh = embs_ref.shape
    V = V1 - 1
    A = wk_ref.shape[1]
    R = _UB * cp
    bf16 = jnp.bfloat16
    f32 = jnp.float32
    inv_sqrt_a = 1.0 / math.sqrt(A)

    embs = embs_ref[...]
    embs_bf = embs.astype(bf16)

    qac = jnp.dot(enc_ref[0].astype(bf16), wqa_ref[...].astype(bf16),
                  preferred_element_type=f32) + bqa_ref[...]

    dec8 = dec_ref[0]
    onehot_dec = (dec8 == jax.lax.broadcasted_iota(jnp.int32, (_UB, V), 1)
                  ).astype(f32).astype(bf16)
    semantic = jnp.dot(onehot_dec, embs_bf[:V, :],
                       preferred_element_type=f32)
    qse8 = jnp.dot(semantic.astype(bf16), wqs_ref[...].astype(bf16),
                   preferred_element_type=f32) + bqs_ref[...]

    idx_col = idxc_ref[0].reshape(R, 1)
    idx_pad = idxp_ref[0]
    c_col = jax.lax.broadcasted_iota(jnp.int32, (_UB, cp, 1),
                                     1).reshape(R, 1)
    valid = (c_col < C) & (idx_col >= 0)

    iota_v = jax.lax.broadcasted_iota(jnp.int32, (R, V1), 1)
    G_f = jnp.where(idx_col == iota_v, 1.0, 0.0)
    idx_row_exp = jnp.broadcast_to(idx_pad[:, None, :],
                                   (_UB, cp, cp)).reshape(R, cp)
    lane_c = jax.lax.broadcasted_iota(jnp.int32, (R, cp), 1)
    has_later = jnp.any((idx_col == idx_row_exp) & (lane_c > c_col),
                        axis=1, keepdims=True)
    G = G_f.astype(bf16)
    S = jnp.where(has_later | ~valid, 0.0, G_f).astype(bf16)

    values = jnp.dot(G, embs_bf, preferred_element_type=f32)
    values_bf = values.astype(bf16)
    keys = jnp.dot(values_bf, wk_ref[...].astype(bf16),
                   preferred_element_type=f32) + bk_ref[...]

    qse_exp = jnp.broadcast_to(qse8[:, None, :],
                               (_UB, cp, A)).reshape(R, A)
    rowdot = jnp.sum(qse_exp * keys, axis=1, keepdims=True)
    pen = jnp.where(valid, 0.0, -1.0e9)
    k_aug = jnp.concatenate([keys, rowdot, pen],
                            axis=1).astype(bf16)
    ones_t = jnp.full((T, 2), 1.0, dtype=f32)
    q_aug = jnp.concatenate([qac, ones_t], axis=1).astype(bf16)
    qk = jax.lax.dot_general(q_aug, k_aug, (((1,), (1,)), ((), ())),
                             preferred_element_type=f32)
    logits = qk * inv_sqrt_a

    m = jnp.max(logits, axis=1, keepdims=True)
    e = jnp.exp(logits - m)
    seg_id = jax.lax.broadcasted_iota(jnp.int32, (R, _UB), 0) // cp
    mseg = (seg_id == jax.lax.broadcasted_iota(jnp.int32, (R, _UB), 1)
            ).astype(f32)
    seg_sum = jnp.dot(e, mseg, preferred_element_type=f32)
    denom = jax.lax.dot_general(seg_sum, mseg, (((1,), (1,)), ((), ())),
                                preferred_element_type=f32)
    atten = e / (denom + 1e-30)

    mdiag = (jax.lax.broadcasted_iota(jnp.int32, (_UB, R), 1) // cp ==
             jax.lax.broadcasted_iota(jnp.int32, (_UB, R), 0)
             ).astype(f32).astype(bf16)
    a_big = (jnp.broadcast_to(atten.astype(bf16)[:, None, :], (T, _UB, R))
             * mdiag[None]).reshape(T * _UB, R)

    x8 = jnp.dot(a_big, values_bf, preferred_element_type=f32)
    h_ref[0] = x8.reshape(T, _UB, Dh)
    d8 = jnp.dot(x8.astype(bf16), wd_ref[...].astype(bf16),
                 preferred_element_type=f32) + bd_ref[...]
    db_ref[0] = d8.reshape(T, _UB, wd_ref.shape[1])
    p8 = jnp.dot(a_big, S, preferred_element_type=f32)
    ptr_ref[0] = p8.reshape(T, _UB, V1)


def kernel(encoder_out, decoder_in, masks_mat, dec_embed_weight, ooKB_weight,
           Wqa, bqa, Wqs, bqs, Wk, bk, Wd, bd):
    B, T, Eh = encoder_out.shape
    U = decoder_in.shape[1]
    C = masks_mat.shape[2]
    V, Dh = dec_embed_weight.shape
    A = Wk.shape[1]
    J = Wd.shape[1]
    V1 = V + 1
    f32 = jnp.float32

    embs = jnp.concatenate([dec_embed_weight, ooKB_weight], axis=0)

    nu = pl.cdiv(U, _UB)
    u_pad = nu * _UB
    cp = pl.cdiv(C, 8) * 8
    masks_row = masks_mat.astype(jnp.int32)
    masks_pad = jnp.pad(masks_row, ((0, 0), (0, u_pad - U), (0, cp - C)),
                        constant_values=-1)
    masks_pad4 = masks_pad[..., None]
    dec3 = jnp.pad(decoder_in.astype(jnp.int32),
                   ((0, 0), (0, u_pad - U)))[..., None]

    grid = (B, nu)
    ptr, h_ptr, dbias = pl.pallas_call(
        functools.partial(_main, C=C),
        grid=grid,
        in_specs=[
            pl.BlockSpec((1, T, Eh), lambda b, u: (b, 0, 0)),
            pl.BlockSpec((1, _UB, 1), lambda b, u: (b, u, 0)),
            pl.BlockSpec((1, _UB, cp), lambda b, u: (b, u, 0)),
            pl.BlockSpec((1, _UB, cp, 1), lambda b, u: (b, u, 0, 0)),
            pl.BlockSpec((V1, Dh), lambda b, u: (0, 0)),
            pl.BlockSpec((Eh, A), lambda b, u: (0, 0)),
            pl.BlockSpec((1, A), lambda b, u: (0, 0)),
            pl.BlockSpec((Dh, A), lambda b, u: (0, 0)),
            pl.BlockSpec((1, A), lambda b, u: (0, 0)),
            pl.BlockSpec((Dh, A), lambda b, u: (0, 0)),
            pl.BlockSpec((1, A), lambda b, u: (0, 0)),
            pl.BlockSpec((Dh, J), lambda b, u: (0, 0)),
            pl.BlockSpec((1, J), lambda b, u: (0, 0)),
        ],
        out_specs=[
            pl.BlockSpec((1, T, _UB, V1), lambda b, u: (b, 0, u, 0)),
            pl.BlockSpec((1, T, _UB, Dh), lambda b, u: (b, 0, u, 0)),
            pl.BlockSpec((1, T, _UB, J), lambda b, u: (b, 0, u, 0)),
        ],
        out_shape=(
            jax.ShapeDtypeStruct((B, T, U, V1), f32),
            jax.ShapeDtypeStruct((B, T, U, Dh), f32),
            jax.ShapeDtypeStruct((B, T, U, J), f32),
        ),
    )(encoder_out, dec3, masks_pad, masks_pad4, embs,
      Wqa, bqa.reshape(1, A), Wqs, bqs.reshape(1, A), Wk, bk.reshape(1, A),
      Wd, bd.reshape(1, J))

    return (ptr, h_ptr, dbias)
